# baseline (device time: 95202 ns/iter reference)
import jax
import jax.numpy as jnp
from jax import lax
from jax.experimental import pallas as pl
from jax.experimental.pallas import tpu as pltpu

N_DEV = 8
M_PER = 1024
H = M_PER // 2
K = 4
SB = H // K
D = 1024


def kernel(partial, gamma):
    gamma2 = gamma.reshape(1, D)

    def body(p_ref, g_ref, out_ref,
             send_r, recv_r, send_l, recv_l, stage_r, stage_l,
             ssem_r, rsem_r, ssem_l, rsem_l, csem_r, csem_l):
        my = lax.axis_index("i")
        left = lax.rem(my + N_DEV - 1, N_DEV)
        right = lax.rem(my + 1, N_DEV)

        barrier_sem = pltpu.get_barrier_semaphore()
        for nbr in (left, right):
            pl.semaphore_signal(
                barrier_sem, inc=1,
                device_id=(nbr,), device_id_type=pl.DeviceIdType.MESH,
            )
        pl.semaphore_wait(barrier_sem, 2)

        def fetch_top(c, slot):
            return pltpu.make_async_copy(
                p_ref.at[0, pl.ds(c * M_PER, H), :],
                stage_r.at[slot], csem_r.at[slot],
            )

        def fetch_bot(c, slot):
            return pltpu.make_async_copy(
                p_ref.at[0, pl.ds(c * M_PER + H, H), :],
                stage_l.at[slot], csem_l.at[slot],
            )

        def rdma(h, k, sbuf, rbuf, ssem, rsem, dev):
            return pltpu.make_async_remote_copy(
                src_ref=sbuf.at[h, pl.ds(k * SB, SB), :],
                dst_ref=rbuf.at[h, pl.ds(k * SB, SB), :],
                send_sem=ssem.at[h, k], recv_sem=rsem.at[h, k],
                device_id=(dev,), device_id_type=pl.DeviceIdType.MESH,
            )

        cp_r = fetch_top(left, 0)
        cp_l = fetch_bot(right, 0)
        cp_r.start()
        cp_l.start()
        cp_r.wait()
        send_r[0] = stage_r[0].astype(jnp.bfloat16)
        for k in range(K):
            rdma(0, k, send_r, recv_r, ssem_r, rsem_r, right).start()
        cp_l.wait()
        send_l[0] = stage_l[0].astype(jnp.bfloat16)
        for k in range(K):
            rdma(0, k, send_l, recv_l, ssem_l, rsem_l, left).start()
        fetch_top(lax.rem(my + 2 * N_DEV - 2, N_DEV), 1).start()
        fetch_bot(lax.rem(my + 2, N_DEV), 1).start()

        for h in range(N_DEV - 1):
            acc_slot = (h + 1) % 2
            pltpu.make_async_copy(
                stage_r.at[acc_slot], stage_r.at[acc_slot], csem_r.at[acc_slot]
            ).wait()
            pltpu.make_async_copy(
                stage_l.at[acc_slot], stage_l.at[acc_slot], csem_l.at[acc_slot]
            ).wait()
            if h < N_DEV - 2:
                fetch_top(lax.rem(my + 2 * N_DEV - h - 3, N_DEV), h % 2).start()
                fetch_bot(lax.rem(my + h + 3, N_DEV), h % 2).start()
            for k in range(K):
                ds_k = pl.ds(k * SB, SB)
                rdma(h, k, send_r, recv_r, ssem_r, rsem_r, right).wait()
                if h < N_DEV - 2:
                    send_r[h + 1, ds_k, :] = (
                        recv_r[h, ds_k, :]
                        + stage_r[acc_slot, ds_k, :].astype(jnp.bfloat16)
                    )
                    rdma(h + 1, k, send_r, recv_r, ssem_r, rsem_r, right).start()
                else:
                    y = (recv_r[h, ds_k, :].astype(jnp.float32)
                         + stage_r[acc_slot, ds_k, :])
                    rms = jnp.sqrt(
                        jnp.mean(y * y, axis=-1, keepdims=True) + 1e-6)
                    out_ref[pl.ds(k * SB, SB), :] = y / rms * g_ref[...]
                rdma(h, k, send_l, recv_l, ssem_l, rsem_l, left).wait()
                if h < N_DEV - 2:
                    send_l[h + 1, ds_k, :] = (
                        recv_l[h, ds_k, :]
                        + stage_l[acc_slot, ds_k, :].astype(jnp.bfloat16)
                    )
                    rdma(h + 1, k, send_l, recv_l, ssem_l, rsem_l, left).start()
                else:
                    y = (recv_l[h, ds_k, :].astype(jnp.float32)
                         + stage_l[acc_slot, ds_k, :])
                    rms = jnp.sqrt(
                        jnp.mean(y * y, axis=-1, keepdims=True) + 1e-6)
                    out_ref[pl.ds(H + k * SB, SB), :] = y / rms * g_ref[...]

    return pl.pallas_call(
        body,
        out_shape=jax.ShapeDtypeStruct((M_PER, D), jnp.float32),
        in_specs=[
            pl.BlockSpec(memory_space=pl.ANY),
            pl.BlockSpec(memory_space=pltpu.VMEM),
        ],
        out_specs=pl.BlockSpec(memory_space=pltpu.VMEM),
        scratch_shapes=[
            pltpu.VMEM((N_DEV - 1, H, D), jnp.bfloat16),
            pltpu.VMEM((N_DEV - 1, H, D), jnp.bfloat16),
            pltpu.VMEM((N_DEV - 1, H, D), jnp.bfloat16),
            pltpu.VMEM((N_DEV - 1, H, D), jnp.bfloat16),
            pltpu.VMEM((2, H, D), jnp.float32),
            pltpu.VMEM((2, H, D), jnp.float32),
            pltpu.SemaphoreType.DMA((N_DEV - 1, K)),
            pltpu.SemaphoreType.DMA((N_DEV - 1, K)),
            pltpu.SemaphoreType.DMA((N_DEV - 1, K)),
            pltpu.SemaphoreType.DMA((N_DEV - 1, K)),
            pltpu.SemaphoreType.DMA((2,)),
            pltpu.SemaphoreType.DMA((2,)),
        ],
        compiler_params=pltpu.CompilerParams(
            collective_id=0,
            vmem_limit_bytes=100 * 1024 * 1024,
        ),
    )(partial, gamma2)


# device time: 94530 ns/iter; 1.0071x vs baseline; 1.0071x over previous
import jax
import jax.numpy as jnp
from jax import lax
from jax.experimental import pallas as pl
from jax.experimental.pallas import tpu as pltpu

N_DEV = 8
M_PER = 1024
H = M_PER // 2
K = 4
SB = H // K
D = 1024


def kernel(partial, gamma):
    gamma2 = gamma.reshape(1, D)

    def body(p_ref, g_ref, out_ref,
             send_r, recv_r, send_l, recv_l, stage_r, stage_l,
             ssem_r, rsem_r, ssem_l, rsem_l, csem_r, csem_l):
        my = lax.axis_index("i")

        def ring_id(t):
            return jnp.where(t < 4, t, 11 - t).astype(my.dtype)

        r = ring_id(my)
        left = ring_id(lax.rem(r + N_DEV - 1, N_DEV))
        right = ring_id(lax.rem(r + 1, N_DEV))

        barrier_sem = pltpu.get_barrier_semaphore()
        for nbr in (left, right):
            pl.semaphore_signal(
                barrier_sem, inc=1,
                device_id=(nbr,), device_id_type=pl.DeviceIdType.MESH,
            )
        pl.semaphore_wait(barrier_sem, 2)

        def fetch_top(c, slot):
            return pltpu.make_async_copy(
                p_ref.at[0, pl.ds(c * M_PER, H), :],
                stage_r.at[slot], csem_r.at[slot],
            )

        def fetch_bot(c, slot):
            return pltpu.make_async_copy(
                p_ref.at[0, pl.ds(c * M_PER + H, H), :],
                stage_l.at[slot], csem_l.at[slot],
            )

        def rdma(h, k, sbuf, rbuf, ssem, rsem, dev):
            return pltpu.make_async_remote_copy(
                src_ref=sbuf.at[h, pl.ds(k * SB, SB), :],
                dst_ref=rbuf.at[h, pl.ds(k * SB, SB), :],
                send_sem=ssem.at[h, k], recv_sem=rsem.at[h, k],
                device_id=(dev,), device_id_type=pl.DeviceIdType.MESH,
            )

        cp_r = fetch_top(left, 0)
        cp_l = fetch_bot(right, 0)
        cp_r.start()
        cp_l.start()
        cp_r.wait()
        send_r[0] = stage_r[0].astype(jnp.bfloat16)
        for k in range(K):
            rdma(0, k, send_r, recv_r, ssem_r, rsem_r, right).start()
        cp_l.wait()
        send_l[0] = stage_l[0].astype(jnp.bfloat16)
        for k in range(K):
            rdma(0, k, send_l, recv_l, ssem_l, rsem_l, left).start()
        fetch_top(ring_id(lax.rem(r + 2 * N_DEV - 2, N_DEV)), 1).start()
        fetch_bot(ring_id(lax.rem(r + 2, N_DEV)), 1).start()

        for h in range(N_DEV - 1):
            acc_slot = (h + 1) % 2
            pltpu.make_async_copy(
                stage_r.at[acc_slot], stage_r.at[acc_slot], csem_r.at[acc_slot]
            ).wait()
            pltpu.make_async_copy(
                stage_l.at[acc_slot], stage_l.at[acc_slot], csem_l.at[acc_slot]
            ).wait()
            if h < N_DEV - 2:
                fetch_top(ring_id(lax.rem(r + 2 * N_DEV - h - 3, N_DEV)), h % 2).start()
                fetch_bot(ring_id(lax.rem(r + h + 3, N_DEV)), h % 2).start()
            for k in range(K):
                ds_k = pl.ds(k * SB, SB)
                rdma(h, k, send_r, recv_r, ssem_r, rsem_r, right).wait()
                if h < N_DEV - 2:
                    send_r[h + 1, ds_k, :] = (
                        recv_r[h, ds_k, :]
                        + stage_r[acc_slot, ds_k, :].astype(jnp.bfloat16)
                    )
                    rdma(h + 1, k, send_r, recv_r, ssem_r, rsem_r, right).start()
                else:
                    y = (recv_r[h, ds_k, :].astype(jnp.float32)
                         + stage_r[acc_slot, ds_k, :])
                    rms = jnp.sqrt(
                        jnp.mean(y * y, axis=-1, keepdims=True) + 1e-6)
                    out_ref[pl.ds(k * SB, SB), :] = y / rms * g_ref[...]
                rdma(h, k, send_l, recv_l, ssem_l, rsem_l, left).wait()
                if h < N_DEV - 2:
                    send_l[h + 1, ds_k, :] = (
                        recv_l[h, ds_k, :]
                        + stage_l[acc_slot, ds_k, :].astype(jnp.bfloat16)
                    )
                    rdma(h + 1, k, send_l, recv_l, ssem_l, rsem_l, left).start()
                else:
                    y = (recv_l[h, ds_k, :].astype(jnp.float32)
                         + stage_l[acc_slot, ds_k, :])
                    rms = jnp.sqrt(
                        jnp.mean(y * y, axis=-1, keepdims=True) + 1e-6)
                    out_ref[pl.ds(H + k * SB, SB), :] = y / rms * g_ref[...]

    return pl.pallas_call(
        body,
        out_shape=jax.ShapeDtypeStruct((M_PER, D), jnp.float32),
        in_specs=[
            pl.BlockSpec(memory_space=pl.ANY),
            pl.BlockSpec(memory_space=pltpu.VMEM),
        ],
        out_specs=pl.BlockSpec(memory_space=pltpu.VMEM),
        scratch_shapes=[
            pltpu.VMEM((N_DEV - 1, H, D), jnp.bfloat16),
            pltpu.VMEM((N_DEV - 1, H, D), jnp.bfloat16),
            pltpu.VMEM((N_DEV - 1, H, D), jnp.bfloat16),
            pltpu.VMEM((N_DEV - 1, H, D), jnp.bfloat16),
            pltpu.VMEM((2, H, D), jnp.float32),
            pltpu.VMEM((2, H, D), jnp.float32),
            pltpu.SemaphoreType.DMA((N_DEV - 1, K)),
            pltpu.SemaphoreType.DMA((N_DEV - 1, K)),
            pltpu.SemaphoreType.DMA((N_DEV - 1, K)),
            pltpu.SemaphoreType.DMA((N_DEV - 1, K)),
            pltpu.SemaphoreType.DMA((2,)),
            pltpu.SemaphoreType.DMA((2,)),
        ],
        compiler_params=pltpu.CompilerParams(
            collective_id=0,
            vmem_limit_bytes=100 * 1024 * 1024,
        ),
    )(partial, gamma2)


# device time: 93335 ns/iter; 1.0200x vs baseline; 1.0128x over previous
import jax
import jax.numpy as jnp
from jax import lax
from jax.experimental import pallas as pl
from jax.experimental.pallas import tpu as pltpu

N_DEV = 8
M_PER = 1024
H = M_PER // 2
K = 4
SB = H // K
D = 1024


def kernel(partial, gamma):
    gamma2 = gamma.reshape(1, D)

    def body(p_ref, g_ref, out_ref,
             send_r, recv_r, send_l, recv_l, stage_r, stage_l,
             ssem_r, rsem_r, ssem_l, rsem_l, csem_r, csem_l):
        my = lax.axis_index("i")

        def ring_id(t):
            return jnp.where(t < 4, t, 11 - t).astype(my.dtype)

        r = ring_id(my)
        left = ring_id(lax.rem(r + N_DEV - 1, N_DEV))
        right = ring_id(lax.rem(r + 1, N_DEV))

        def fetch_top(c, slot):
            return pltpu.make_async_copy(
                p_ref.at[0, pl.ds(c * M_PER, H), :],
                stage_r.at[slot], csem_r.at[slot],
            )

        def fetch_bot(c, slot):
            return pltpu.make_async_copy(
                p_ref.at[0, pl.ds(c * M_PER + H, H), :],
                stage_l.at[slot], csem_l.at[slot],
            )

        def rdma(h, k, sbuf, rbuf, ssem, rsem, dev):
            return pltpu.make_async_remote_copy(
                src_ref=sbuf.at[h, pl.ds(k * SB, SB), :],
                dst_ref=rbuf.at[h, pl.ds(k * SB, SB), :],
                send_sem=ssem.at[h, k], recv_sem=rsem.at[h, k],
                device_id=(dev,), device_id_type=pl.DeviceIdType.MESH,
            )

        cp_r = fetch_top(left, 0)
        cp_l = fetch_bot(right, 0)
        cp_r.start()
        cp_l.start()

        barrier_sem = pltpu.get_barrier_semaphore()
        for nbr in (left, right):
            pl.semaphore_signal(
                barrier_sem, inc=1,
                device_id=(nbr,), device_id_type=pl.DeviceIdType.MESH,
            )
        pl.semaphore_wait(barrier_sem, 2)

        cp_r.wait()
        send_r[0] = stage_r[0].astype(jnp.bfloat16)
        for k in range(K):
            rdma(0, k, send_r, recv_r, ssem_r, rsem_r, right).start()
        cp_l.wait()
        send_l[0] = stage_l[0].astype(jnp.bfloat16)
        for k in range(K):
            rdma(0, k, send_l, recv_l, ssem_l, rsem_l, left).start()
        fetch_top(ring_id(lax.rem(r + 2 * N_DEV - 2, N_DEV)), 1).start()
        fetch_bot(ring_id(lax.rem(r + 2, N_DEV)), 1).start()

        for h in range(N_DEV - 1):
            acc_slot = (h + 1) % 2
            pltpu.make_async_copy(
                stage_r.at[acc_slot], stage_r.at[acc_slot], csem_r.at[acc_slot]
            ).wait()
            pltpu.make_async_copy(
                stage_l.at[acc_slot], stage_l.at[acc_slot], csem_l.at[acc_slot]
            ).wait()
            if h < N_DEV - 2:
                fetch_top(ring_id(lax.rem(r + 2 * N_DEV - h - 3, N_DEV)), h % 2).start()
                fetch_bot(ring_id(lax.rem(r + h + 3, N_DEV)), h % 2).start()
            for k in range(K):
                ds_k = pl.ds(k * SB, SB)
                rdma(h, k, send_r, recv_r, ssem_r, rsem_r, right).wait()
                if h < N_DEV - 2:
                    send_r[h + 1, ds_k, :] = (
                        recv_r[h, ds_k, :]
                        + stage_r[acc_slot, ds_k, :].astype(jnp.bfloat16)
                    )
                    rdma(h + 1, k, send_r, recv_r, ssem_r, rsem_r, right).start()
                else:
                    y = (recv_r[h, ds_k, :].astype(jnp.float32)
                         + stage_r[acc_slot, ds_k, :])
                    rms = jnp.sqrt(
                        jnp.mean(y * y, axis=-1, keepdims=True) + 1e-6)
                    out_ref[pl.ds(k * SB, SB), :] = y / rms * g_ref[...]
                rdma(h, k, send_l, recv_l, ssem_l, rsem_l, left).wait()
                if h < N_DEV - 2:
                    send_l[h + 1, ds_k, :] = (
                        recv_l[h, ds_k, :]
                        + stage_l[acc_slot, ds_k, :].astype(jnp.bfloat16)
                    )
                    rdma(h + 1, k, send_l, recv_l, ssem_l, rsem_l, left).start()
                else:
                    y = (recv_l[h, ds_k, :].astype(jnp.float32)
                         + stage_l[acc_slot, ds_k, :])
                    rms = jnp.sqrt(
                        jnp.mean(y * y, axis=-1, keepdims=True) + 1e-6)
                    out_ref[pl.ds(H + k * SB, SB), :] = y / rms * g_ref[...]

    return pl.pallas_call(
        body,
        out_shape=jax.ShapeDtypeStruct((M_PER, D), jnp.float32),
        in_specs=[
            pl.BlockSpec(memory_space=pl.ANY),
            pl.BlockSpec(memory_space=pltpu.VMEM),
        ],
        out_specs=pl.BlockSpec(memory_space=pltpu.VMEM),
        scratch_shapes=[
            pltpu.VMEM((N_DEV - 1, H, D), jnp.bfloat16),
            pltpu.VMEM((N_DEV - 1, H, D), jnp.bfloat16),
            pltpu.VMEM((N_DEV - 1, H, D), jnp.bfloat16),
            pltpu.VMEM((N_DEV - 1, H, D), jnp.bfloat16),
            pltpu.VMEM((2, H, D), jnp.float32),
            pltpu.VMEM((2, H, D), jnp.float32),
            pltpu.SemaphoreType.DMA((N_DEV - 1, K)),
            pltpu.SemaphoreType.DMA((N_DEV - 1, K)),
            pltpu.SemaphoreType.DMA((N_DEV - 1, K)),
            pltpu.SemaphoreType.DMA((N_DEV - 1, K)),
            pltpu.SemaphoreType.DMA((2,)),
            pltpu.SemaphoreType.DMA((2,)),
        ],
        compiler_params=pltpu.CompilerParams(
            collective_id=0,
            vmem_limit_bytes=100 * 1024 * 1024,
        ),
    )(partial, gamma2)
